# baseline (speedup 1.0000x reference)
import jax
import jax.numpy as jnp
from jax.experimental import pallas as pl
from jax.experimental.pallas import tpu as pltpu

_BLOCK = 256


def _embed_block(idx_ref, t0_ref, dt_ref, out_ref):
    w = idx_ref[...].astype(jnp.float32)[:, None, :]
    t0 = t0_ref[...][None, :, :]
    dt = dt_ref[...][None, :, :]
    out_ref[...] = t0 + w * dt


def kernel(inputs, table):
    rows, cols = inputs.shape
    dim = table.shape[1]
    idx_t = inputs.T
    t0 = table[0].reshape(dim, 1)
    dt = (table[1] - table[0]).reshape(dim, 1)
    grid = (rows // _BLOCK,)
    out_t = pl.pallas_call(
        _embed_block,
        grid=grid,
        in_specs=[
            pl.BlockSpec((cols, _BLOCK), lambda i: (0, i)),
            pl.BlockSpec((dim, 1), lambda i: (0, 0)),
            pl.BlockSpec((dim, 1), lambda i: (0, 0)),
        ],
        out_specs=pl.BlockSpec((cols, dim, _BLOCK), lambda i: (0, 0, i)),
        out_shape=jax.ShapeDtypeStruct((cols, dim, rows), jnp.float32),
    )(idx_t, t0, dt)
    return out_t.transpose(2, 0, 1)


# speedup vs baseline: 1.0452x; 1.0452x over previous
import jax
import jax.numpy as jnp
from jax.experimental import pallas as pl
from jax.experimental.pallas import tpu as pltpu

_BLOCK = 768


def _embed_block(idx_ref, t0_ref, dt_ref, out_ref):
    w = idx_ref[...].astype(jnp.float32)[:, None, :]
    t0 = t0_ref[...][None, :, :]
    dt = dt_ref[...][None, :, :]
    out_ref[...] = t0 + w * dt


def kernel(inputs, table):
    rows, cols = inputs.shape
    dim = table.shape[1]
    idx_t = inputs.T
    t0 = table[0].reshape(dim, 1)
    dt = (table[1] - table[0]).reshape(dim, 1)
    grid = (rows // _BLOCK,)
    out_t = pl.pallas_call(
        _embed_block,
        grid=grid,
        in_specs=[
            pl.BlockSpec((cols, _BLOCK), lambda i: (0, i)),
            pl.BlockSpec((dim, 1), lambda i: (0, 0)),
            pl.BlockSpec((dim, 1), lambda i: (0, 0)),
        ],
        out_specs=pl.BlockSpec((cols, dim, _BLOCK), lambda i: (0, 0, i)),
        out_shape=jax.ShapeDtypeStruct((cols, dim, rows), jnp.float32),
    )(idx_t, t0, dt)
    return out_t.transpose(2, 0, 1)
